# baseline (speedup 1.0000x reference)
import functools

import jax
import jax.numpy as jnp
from jax.experimental import pallas as pl

def _fused_kernel(seq_ref, maskf_ref, tgt_ref, pos_ref, w1i_ref, b1i_ref,
                  w2i_ref, b2i_ref, w1e_ref, b1e_ref, w2e_ref, b2e_ref,
                  gam_ref, bet_ref, out_ref, *, K, rank_chunk):
    Bb, L, D = seq_ref.shape
    H = w1i_ref.shape[1]
    OUT = w2e_ref.shape[1]

    seq3 = seq_ref[...]
    seq2 = seq3.reshape(Bb * L, D)
    pos = pos_ref[...]
    tgt = tgt_ref[...]

    w1i = w1i_ref[...]
    wa, wb, wc = w1i[:D], w1i[D:2 * D], w1i[2 * D:]
    seqT = jnp.transpose(seq3, (0, 2, 1))
    wa_b = jnp.broadcast_to(wa[None], (Bb, D, H))
    hT = jax.lax.dot_general(wa_b, seqT, (((1,), (1,)), ((0,), (0,))),
                             preferred_element_type=jnp.float32)
    pos_wbT = jnp.transpose(jnp.dot(pos, wb, preferred_element_type=jnp.float32)
                            + b1i_ref[...])
    tgt_wc = jnp.dot(tgt, wc, preferred_element_type=jnp.float32)
    hT = jnp.maximum(hT + pos_wbT[None, :, :] + tgt_wc[:, :, None], 0.0)
    scores = (jnp.sum(hT * w2i_ref[...].reshape(1, H, 1), axis=1)
              + b2i_ref[0, 0])
    scores = jnp.where(maskf_ref[...] == 0.0, -1e9, scores)

    tri_f = jnp.where(jax.lax.broadcasted_iota(jnp.int32, (L, L), 0)
                      < jax.lax.broadcasted_iota(jnp.int32, (L, L), 1),
                      1.0, 0.0)
    maskz = jnp.where(maskf_ref[...] == 0.0, 1.0, 0.0)
    mcum = jax.lax.dot_general(maskz, tri_f, (((1,), (0,)), ((), ())),
                               preferred_element_type=jnp.float32)
    tie_corr = maskz * mcum
    sel_parts = []
    for c0 in range(0, Bb, rank_chunk):
        s_c = scores[c0:c0 + rank_chunk]
        a = s_c[:, :, None]
        b = s_c[:, None, :]
        rank = tie_corr[c0:c0 + rank_chunk]
        for j0 in range(0, L, 40):
            a_g = a[:, j0:j0 + min(40, L - j0), :]
            rank = rank + jnp.sum(jnp.where(a_g > b, 1.0, 0.0), axis=1)
        sel_parts.append(rank < K)
    selected = jnp.concatenate(sel_parts, axis=0)

    m = jnp.max(scores, axis=1, keepdims=True)
    w = jnp.where(selected, jnp.exp(scores - m), 0.0)
    attn = w / jnp.sum(w, axis=1, keepdims=True)

    w1e = w1e_ref[...]
    we_s, we_p = w1e[:D], w1e[D:]
    pos_wep = (jnp.dot(pos, we_p, preferred_element_type=jnp.float32)
               + b1e_ref[...])
    h2 = jnp.dot(seq2, we_s, preferred_element_type=jnp.float32).reshape(Bb, L, H)
    h2 = jnp.maximum(h2 + pos_wep[None, :, :], 0.0)
    ph = jax.lax.dot_general(attn, h2, (((1,), (1,)), ((0,), (0,))),
                             preferred_element_type=jnp.float32)
    pooled = jnp.dot(ph, w2e_ref[...],
                     preferred_element_type=jnp.float32) + b2e_ref[...]

    mean = jnp.mean(pooled, axis=1, keepdims=True)
    cent = pooled - mean
    var = jnp.mean(cent * cent, axis=1, keepdims=True)
    out_ref[...] = cent / jnp.sqrt(var + 1e-5) * gam_ref[...] + bet_ref[...]


def kernel(sequence_emb, mask, target_emb, pos_table, W1_imp, b1_imp, W2_imp,
           b2_imp, W1_enc, b1_enc, W2_enc, b2_enc, ln_gamma, ln_beta):
    B, L, D = sequence_emb.shape
    H = W1_imp.shape[1]
    OUT = W2_enc.shape[1]
    K = min(50, L)

    maskf = mask.astype(jnp.float32)
    pos = pos_table[:L]

    b1i = b1_imp.reshape(1, H)
    w2i = W2_imp.reshape(1, H)
    b2i = b2_imp.reshape(1, 1)
    b1e = b1_enc.reshape(1, H)
    b2e = b2_enc.reshape(1, OUT)
    gam = ln_gamma.reshape(1, OUT)
    bet = ln_beta.reshape(1, OUT)

    Bb = 128
    grid = (B // Bb,)

    full = lambda arr: pl.BlockSpec(arr.shape, lambda i: (0,) * arr.ndim)

    out = pl.pallas_call(
        functools.partial(_fused_kernel, K=K, rank_chunk=8),
        grid=grid,
        in_specs=[
            pl.BlockSpec((Bb, L, D), lambda i: (i, 0, 0)),
            pl.BlockSpec((Bb, L), lambda i: (i, 0)),
            pl.BlockSpec((Bb, D), lambda i: (i, 0)),
            full(pos),
            full(W1_imp),
            full(b1i),
            full(w2i),
            full(b2i),
            full(W1_enc),
            full(b1e),
            full(W2_enc),
            full(b2e),
            full(gam),
            full(bet),
        ],
        out_specs=pl.BlockSpec((Bb, OUT), lambda i: (i, 0)),
        out_shape=jax.ShapeDtypeStruct((B, OUT), jnp.float32),
    )(sequence_emb, maskf, target_emb, pos, W1_imp, b1i, w2i, b2i,
      W1_enc, b1e, W2_enc, b2e, gam, bet)
    return out


# speedup vs baseline: 1.0236x; 1.0236x over previous
import functools

import jax
import jax.numpy as jnp
from jax.experimental import pallas as pl

def _fused_kernel(seq_ref, maskf_ref, tgt_ref, pos_ref, w1i_ref, b1i_ref,
                  w2i_ref, b2i_ref, w1e_ref, b1e_ref, w2e_ref, b2e_ref,
                  gam_ref, bet_ref, out_ref, *, K, rank_chunk):
    Bb, L, D = seq_ref.shape
    H = w1i_ref.shape[1]
    OUT = w2e_ref.shape[1]

    seq3 = seq_ref[...]
    seq2 = seq3.reshape(Bb * L, D)
    pos = pos_ref[...]
    tgt = tgt_ref[...]

    w1i = w1i_ref[...]
    wa, wb, wc = w1i[:D], w1i[D:2 * D], w1i[2 * D:]
    seqT = jnp.transpose(seq3, (0, 2, 1))
    wa_b = jnp.broadcast_to(wa[None], (Bb, D, H))
    hT = jax.lax.dot_general(wa_b, seqT, (((1,), (1,)), ((0,), (0,))),
                             preferred_element_type=jnp.float32)
    pos_wbT = jnp.transpose(jnp.dot(pos, wb, preferred_element_type=jnp.float32)
                            + b1i_ref[...])
    tgt_wc = jnp.dot(tgt, wc, preferred_element_type=jnp.float32)
    hT = jnp.maximum(hT + pos_wbT[None, :, :] + tgt_wc[:, :, None], 0.0)
    scores = (jnp.sum(hT * w2i_ref[...].reshape(1, H, 1), axis=1)
              + b2i_ref[0, 0])
    scores = jnp.where(maskf_ref[...] == 0.0, -1e9, scores)

    tri_f = jnp.where(jax.lax.broadcasted_iota(jnp.int32, (L, L), 0)
                      < jax.lax.broadcasted_iota(jnp.int32, (L, L), 1),
                      1.0, 0.0)
    maskz = jnp.where(maskf_ref[...] == 0.0, 1.0, 0.0)
    mcum = jax.lax.dot_general(maskz, tri_f, (((1,), (0,)), ((), ())),
                               preferred_element_type=jnp.float32)
    tie_corr = maskz * mcum
    sel_parts = []
    for c0 in range(0, Bb, rank_chunk):
        s_c = scores[c0:c0 + rank_chunk]
        a = s_c[:, :, None]
        b = s_c[:, None, :]
        gt = jnp.where(a > b, 1.0, 0.0)
        rank = jnp.sum(gt, axis=1) + tie_corr[c0:c0 + rank_chunk]
        sel_parts.append(rank < K)
    selected = jnp.concatenate(sel_parts, axis=0)

    m = jnp.max(scores, axis=1, keepdims=True)
    w = jnp.where(selected, jnp.exp(scores - m), 0.0)
    attn = w / jnp.sum(w, axis=1, keepdims=True)

    w1e = w1e_ref[...]
    we_s, we_p = w1e[:D], w1e[D:]
    pos_wep = (jnp.dot(pos, we_p, preferred_element_type=jnp.float32)
               + b1e_ref[...])
    h2 = jnp.dot(seq2, we_s, preferred_element_type=jnp.float32).reshape(Bb, L, H)
    h2 = jnp.maximum(h2 + pos_wep[None, :, :], 0.0)
    ph = jax.lax.dot_general(attn, h2, (((1,), (1,)), ((0,), (0,))),
                             preferred_element_type=jnp.float32)
    pooled = jnp.dot(ph, w2e_ref[...],
                     preferred_element_type=jnp.float32) + b2e_ref[...]

    mean = jnp.mean(pooled, axis=1, keepdims=True)
    cent = pooled - mean
    var = jnp.mean(cent * cent, axis=1, keepdims=True)
    out_ref[...] = cent / jnp.sqrt(var + 1e-5) * gam_ref[...] + bet_ref[...]


def kernel(sequence_emb, mask, target_emb, pos_table, W1_imp, b1_imp, W2_imp,
           b2_imp, W1_enc, b1_enc, W2_enc, b2_enc, ln_gamma, ln_beta):
    B, L, D = sequence_emb.shape
    H = W1_imp.shape[1]
    OUT = W2_enc.shape[1]
    K = min(50, L)

    maskf = mask.astype(jnp.float32)
    pos = pos_table[:L]

    b1i = b1_imp.reshape(1, H)
    w2i = W2_imp.reshape(1, H)
    b2i = b2_imp.reshape(1, 1)
    b1e = b1_enc.reshape(1, H)
    b2e = b2_enc.reshape(1, OUT)
    gam = ln_gamma.reshape(1, OUT)
    bet = ln_beta.reshape(1, OUT)

    Bb = 128
    grid = (B // Bb,)

    full = lambda arr: pl.BlockSpec(arr.shape, lambda i: (0,) * arr.ndim)

    out = pl.pallas_call(
        functools.partial(_fused_kernel, K=K, rank_chunk=8),
        grid=grid,
        in_specs=[
            pl.BlockSpec((Bb, L, D), lambda i: (i, 0, 0)),
            pl.BlockSpec((Bb, L), lambda i: (i, 0)),
            pl.BlockSpec((Bb, D), lambda i: (i, 0)),
            full(pos),
            full(W1_imp),
            full(b1i),
            full(w2i),
            full(b2i),
            full(W1_enc),
            full(b1e),
            full(W2_enc),
            full(b2e),
            full(gam),
            full(bet),
        ],
        out_specs=pl.BlockSpec((Bb, OUT), lambda i: (i, 0)),
        out_shape=jax.ShapeDtypeStruct((B, OUT), jnp.float32),
    )(sequence_emb, maskf, target_emb, pos, W1_imp, b1i, w2i, b2i,
      W1_enc, b1e, W2_enc, b2e, gam, bet)
    return out
